# baseline (speedup 1.0000x reference)
import functools

import jax
import jax.numpy as jnp
from jax import lax
from jax.experimental import pallas as pl
from jax.experimental.pallas import tpu as pltpu
from jax.experimental.pallas import tpu_sc as plsc

EMB_DIM = 32
NBUF = 4


@functools.partial(jax.jit, static_argnums=(2, 3))
def _gather_sc(x_flat, table, B, C):
    NW = 32
    b_per_w = B // NW
    n_chunks = b_per_w // C
    n_groups = n_chunks // NBUF
    mesh = plsc.VectorSubcoreMesh(core_axis_name="c", subcore_axis_name="s")

    @functools.partial(
        pl.kernel,
        mesh=mesh,
        out_type=jax.ShapeDtypeStruct((B, EMB_DIM), jnp.float32),
        scratch_types=[pltpu.VMEM((b_per_w,), jnp.int32)]
        + [pltpu.VMEM((C, EMB_DIM), jnp.float32) for _ in range(NBUF)]
        + [pltpu.SemaphoreType.DMA for _ in range(2 * NBUF)],
        compiler_params=pltpu.CompilerParams(use_tc_tiling_on_sc=False),
    )
    def k(idx_hbm, table_hbm, out_hbm, idx_v, *bufs_and_sems):
        rows = bufs_and_sems[:NBUF]
        sg = bufs_and_sems[NBUF : 2 * NBUF]
        so = bufs_and_sems[2 * NBUF :]
        wid = lax.axis_index("s") * 2 + lax.axis_index("c")
        base = wid * b_per_w
        pltpu.sync_copy(idx_hbm.at[pl.ds(base, b_per_w)], idx_v)

        def gather(i, b):
            pltpu.async_copy(table_hbm.at[idx_v.at[pl.ds(i * C, C)]], rows[b], sg[b])

        def wait_gather(b):
            pltpu.make_async_copy(
                table_hbm.at[idx_v.at[pl.ds(0, C)]], rows[b], sg[b]
            ).wait()

        def store(i, b):
            pltpu.async_copy(rows[b], out_hbm.at[pl.ds(base + i * C, C)], so[b])

        def wait_store(b):
            pltpu.make_async_copy(rows[b], out_hbm.at[pl.ds(base, C)], so[b]).wait()

        for b in range(NBUF):
            gather(b, b)

        def body(g, carry):
            for b in range(NBUF):
                i = g * NBUF + b
                wait_gather(b)
                store(i, b)
                b2 = (b + 2) % NBUF
                j = i - 2

                @pl.when(jnp.logical_and(j >= 0, j + NBUF < n_chunks))
                def _():
                    wait_store(b2)
                    gather(j + NBUF, b2)

            return carry

        lax.fori_loop(0, n_groups, body, 0)

        for i in range(n_chunks - 4, n_chunks):
            wait_store(i % NBUF)

    return k(x_flat, table)


def kernel(x, table):
    B = x.shape[0] * x.shape[1]
    out = _gather_sc(x.reshape(B), table, B, 800)
    return out.reshape(x.shape[0], x.shape[1], EMB_DIM)
